# baseline (speedup 1.0000x reference)
import jax
import jax.numpy as jnp
from jax import lax
from jax.experimental import pallas as pl
from jax.experimental.pallas import tpu as pltpu
from jax.experimental.pallas import tpu_sc as plsc

N = 10000
E = 320000
R = 8
D = 128

NC = 2
NS = 16
NW = NC * NS

C = 128
EPW = E // NW
EPWP = 10240
NCH = EPWP // C
CA = 64
NCHE = EPWP // CA
E2 = 2 * E
TPW = E2 // NW
TPWP = 20480
DCH = TPWP // C
NBLK = 32
BL = E // NBLK
NR = N * R
NRP = 81920
NP = 10240
RPT = NP // NS
PAD_DST = 10200

_MESH = plsc.VectorSubcoreMesh(core_axis_name="c", subcore_axis_name="s",
                               num_cores=NC, num_subcores=NS)
_SC_PARAMS = pltpu.CompilerParams(needs_layout_passes=False)

_BN = 2000
_NB = N // _BN


def _proj_body(x_ref, w_ref, b_ref, o_ref):
    o_ref[...] = jnp.dot(x_ref[...], w_ref[...],
                         preferred_element_type=jnp.float32) + b_ref[...]


def _proj_mm(x, w, b):
    return pl.pallas_call(
        _proj_body,
        grid=(_NB,),
        in_specs=[pl.BlockSpec((_BN, D), lambda i: (i, 0)),
                  pl.BlockSpec((D, D), lambda i: (0, 0)),
                  pl.BlockSpec((1, D), lambda i: (0, 0))],
        out_specs=pl.BlockSpec((_BN, D), lambda i: (i, 0)),
        out_shape=jax.ShapeDtypeStruct((N, D), jnp.float32),
    )(x, w, b)


def _layer_first_body(x_ref, wr_ref, wo_ref, b_ref, y_ref, root_ref):
    r = pl.program_id(1)
    xb = x_ref[...]
    y_ref[...] = jnp.dot(xb, wr_ref[0],
                         preferred_element_type=jnp.float32)[None]

    @pl.when(r == 0)
    def _():
        root_ref[...] = jnp.dot(xb, wo_ref[...],
                                preferred_element_type=jnp.float32) + b_ref[...]


def _layer_mid_body(p_ref, rin_ref, wr_ref, wo_ref, b_ref, y_ref, root_ref):
    r = pl.program_id(1)
    xb = jax.nn.relu(p_ref[0] + p_ref[1] + rin_ref[...])
    y_ref[...] = jnp.dot(xb, wr_ref[0],
                         preferred_element_type=jnp.float32)[None]

    @pl.when(r == 0)
    def _():
        root_ref[...] = jnp.dot(xb, wo_ref[...],
                                preferred_element_type=jnp.float32) + b_ref[...]


def _layer_mm(x_or_p, rin, w_rel, w_root, b, first):
    specs = [
        pl.BlockSpec((D, D), lambda i, r: (0, 0)),
        pl.BlockSpec((1, D), lambda i, r: (0, 0)),
    ]
    wr_spec = pl.BlockSpec((1, D, D), lambda i, r: (r, 0, 0))
    if first:
        in_specs = [pl.BlockSpec((_BN, D), lambda i, r: (i, 0)), wr_spec] + specs
        body = _layer_first_body
        args = (x_or_p, w_rel, w_root, b)
    else:
        in_specs = [pl.BlockSpec((2, _BN, D), lambda i, r: (0, i, 0)),
                    pl.BlockSpec((_BN, D), lambda i, r: (i, 0)),
                    wr_spec] + specs
        body = _layer_mid_body
        args = (x_or_p, rin, w_rel, w_root, b)
    return pl.pallas_call(
        body,
        grid=(_NB, R),
        in_specs=in_specs,
        out_specs=[pl.BlockSpec((1, _BN, D), lambda i, r: (r, i, 0)),
                   pl.BlockSpec((_BN, D), lambda i, r: (i, 0))],
        out_shape=[jax.ShapeDtypeStruct((R, N, D), jnp.float32),
                   jax.ShapeDtypeStruct((N, D), jnp.float32)],
    )(*args)


def _final_body(p_ref, rin_ref, out_ref):
    out_ref[...] = p_ref[0] + p_ref[1] + rin_ref[...]


def _final_mm(p, rin):
    return pl.pallas_call(
        _final_body,
        grid=(_NB,),
        in_specs=[pl.BlockSpec((2, _BN, D), lambda i: (0, i, 0)),
                  pl.BlockSpec((_BN, D), lambda i: (i, 0))],
        out_specs=pl.BlockSpec((_BN, D), lambda i: (i, 0)),
        out_shape=jax.ShapeDtypeStruct((N, D), jnp.float32),
    )(p, rin)


CSL = 2560


def _count1_body(dst3, rel3, counts1, dbuf, rbuf, tbl):
    cid = lax.axis_index("c")
    sid = lax.axis_index("s")
    w = cid * NS + sid
    ones16 = jnp.full((16,), 1.0, jnp.float32)
    z16 = jnp.zeros((16,), jnp.float32)
    pltpu.sync_copy(dst3.at[w], dbuf)
    pltpu.sync_copy(rel3.at[w], rbuf)

    def zero_body(k, _):
        tbl[pl.ds(k * 16, 16)] = z16
        return 0

    lax.fori_loop(0, NRP // 16, zero_body, 0)

    def g_body(g, _):
        r = g // (C // 16)
        c = (g % (C // 16)) * 16
        d16 = dbuf[r, pl.ds(c, 16)]
        r16 = rbuf[r, pl.ds(c, 16)]
        plsc.addupdate_scatter(tbl, [d16 * R + r16], ones16)
        return 0

    lax.fori_loop(0, NCH * (C // 16), g_body, 0)
    pltpu.sync_copy(tbl, counts1.at[w])


def _count2_body(counts1, counts_out, inbuf, obuf, sem):
    cid = lax.axis_index("c")
    sid = lax.axis_index("s")
    w = cid * NS + sid

    def reduce_slice(off, ln):
        for j in range(NW):
            pltpu.async_copy(counts1.at[j, pl.ds(off, ln)],
                             inbuf.at[j, pl.ds(0, ln)], sem)
        for j in range(NW):
            pltpu.make_async_copy(counts1.at[j, pl.ds(off, ln)],
                                  inbuf.at[j, pl.ds(0, ln)], sem).wait()

        def g_body(g, _):
            sl = pl.ds(g * 16, 16)
            a = inbuf[0, sl]
            for j in range(1, NW):
                a = a + inbuf[j, sl]
            obuf[sl] = a
            return 0

        lax.fori_loop(0, ln // 16, g_body, 0)
        pltpu.sync_copy(obuf.at[pl.ds(0, ln)], counts_out.at[pl.ds(off, ln)])

    @pl.when(w < 31)
    def _():
        reduce_slice(w * CSL, CSL)

    @pl.when(w == 31)
    def _():
        reduce_slice(31 * CSL, NR - 31 * CSL)


def _count(dst3, rel3):
    counts1 = pl.kernel(
        _count1_body,
        out_type=jax.ShapeDtypeStruct((NW, NRP), jnp.float32),
        mesh=_MESH,
        compiler_params=_SC_PARAMS,
        scratch_types=[pltpu.VMEM((NCH, C), jnp.int32),
                       pltpu.VMEM((NCH, C), jnp.int32),
                       pltpu.VMEM((NRP,), jnp.float32)],
    )(dst3, rel3)
    return pl.kernel(
        _count2_body,
        out_type=jax.ShapeDtypeStruct((NR,), jnp.float32),
        mesh=_MESH,
        compiler_params=_SC_PARAMS,
        scratch_types=[pltpu.VMEM((NW, CSL), jnp.float32),
                       pltpu.VMEM((CSL,), jnp.float32),
                       pltpu.SemaphoreType.DMA],
    )(counts1)


def _norm_body(dst3, rel3, counts, norm_out, dbuf, rbuf, nbuf, tbl):
    cid = lax.axis_index("c")
    sid = lax.axis_index("s")
    w = cid * NS + sid
    pltpu.sync_copy(counts, tbl.at[pl.ds(0, NR)])
    pltpu.sync_copy(dst3.at[w], dbuf)
    pltpu.sync_copy(rel3.at[w], rbuf)

    G = C // 16

    def g_body(k, _):
        r = k // G
        c = (k % G) * 16
        d16 = dbuf[r, pl.ds(c, 16)]
        r16 = rbuf[r, pl.ds(c, 16)]
        c16 = plsc.load_gather(tbl, [d16 * R + r16])
        nbuf[r, pl.ds(c, 16)] = 1.0 / jnp.maximum(c16, 1.0)
        return 0

    lax.fori_loop(0, NCH * G, g_body, 0)
    pltpu.sync_copy(nbuf, norm_out.at[w])


def _norm(dst3, rel3, counts):
    return pl.kernel(
        _norm_body,
        out_type=jax.ShapeDtypeStruct((NW, NCH, C), jnp.float32),
        mesh=_MESH,
        compiler_params=_SC_PARAMS,
        scratch_types=[pltpu.VMEM((NCH, C), jnp.int32),
                       pltpu.VMEM((NCH, C), jnp.int32),
                       pltpu.VMEM((NCH, C), jnp.float32),
                       pltpu.VMEM((NRP,), jnp.float32)],
    )(dst3, rel3, counts)


def _agg_body(y2, g2d, dst3, norm3, zeros_nd, out,
              gbuf, d2, n2, rows4, acc,
              sg0, sg1, sg2, sg3, ss0, ss1, ss2, ss3):
    cid = lax.axis_index("c")
    sid = lax.axis_index("s")
    w = cid * NS + sid
    base = sid * RPT
    sg = (sg0, sg1, sg2, sg3)
    ss = (ss0, ss1, ss2, ss3)
    pltpu.sync_copy(zeros_nd.at[pl.ds(base, RPT)], acc.at[pl.ds(base, RPT)])
    pltpu.sync_copy(g2d.at[w], gbuf)
    plsc.subcore_barrier()

    def start_g(i, b):
        pltpu.async_copy(y2.at[gbuf.at[pl.ds(i * CA, CA)]], rows4.at[b], sg[b])
        pltpu.async_copy(dst3.at[w, i], d2.at[b], sg[b])
        pltpu.async_copy(norm3.at[w, i], n2.at[b], sg[b])

    def wait_g(b):
        pltpu.make_async_copy(y2.at[gbuf.at[pl.ds(0, CA)]], rows4.at[b],
                              sg[b]).wait()
        pltpu.make_async_copy(dst3.at[w, 0], d2.at[b], sg[b]).wait()
        pltpu.make_async_copy(norm3.at[w, 0], n2.at[b], sg[b]).wait()

    def wait_s(b):
        pltpu.make_async_copy(rows4.at[b], acc.at[d2.at[b]], ss[b]).wait()

    start_g(0, 0)
    start_g(1, 1)

    GA = CA // 16

    def chunk_quad(k, _):
        for b in range(4):
            i = 4 * k + b
            wait_g(b)
            b16 = jnp.full((16,), b, jnp.int32)

            def scale_grp(gi, _):
                e0 = gi * 16
                for t in range(16):
                    e = e0 + t
                    nv = plsc.load_gather(n2, [b16, jnp.broadcast_to(e, (16,))])
                    for j in range(8):
                        sl = pl.ds(j * 16, 16)
                        rows4[b, e, sl] = rows4[b, e, sl] * nv
                return 0

            lax.fori_loop(0, GA, scale_grp, 0)
            pltpu.async_copy(rows4.at[b], acc.at[d2.at[b]], ss[b], add=True)
            b2 = (b + 2) % 4

            @pl.when(i + 2 < NCHE)
            def _():
                @pl.when(i >= 2)
                def _():
                    wait_s(b2)

                start_g(i + 2, b2)
        return 0

    lax.fori_loop(0, NCHE // 4, chunk_quad, 0)
    for b in range(4):
        wait_s(b)
    plsc.subcore_barrier()
    pltpu.sync_copy(acc.at[pl.ds(base, RPT)], out.at[cid, pl.ds(base, RPT)])


def _agg(y2, g2d, dst3, norm3, zeros_nd):
    return pl.kernel(
        _agg_body,
        out_type=jax.ShapeDtypeStruct((NC, NP, D), jnp.float32),
        mesh=_MESH,
        compiler_params=_SC_PARAMS,
        scratch_types=[pltpu.VMEM((EPWP,), jnp.int32),
                       pltpu.VMEM((4, CA), jnp.int32),
                       pltpu.VMEM((4, CA), jnp.float32),
                       pltpu.VMEM((4, CA, D), jnp.float32),
                       pltpu.VMEM_SHARED((NP, D), jnp.float32),
                       pltpu.SemaphoreType.DMA,
                       pltpu.SemaphoreType.DMA,
                       pltpu.SemaphoreType.DMA,
                       pltpu.SemaphoreType.DMA,
                       pltpu.SemaphoreType.DMA,
                       pltpu.SemaphoreType.DMA,
                       pltpu.SemaphoreType.DMA,
                       pltpu.SemaphoreType.DMA],
    )(y2, g2d, dst3, norm3, zeros_nd)


def _dec_body(out_nd, asrc3, adst3, arel3, relv_hbm, scores,
              gbuf, dbuf, rbuf, relv, trows2, drows2, sc2, tsc,
              semg0, semg1, semsc):
    cid = lax.axis_index("c")
    sid = lax.axis_index("s")
    w = cid * NS + sid
    pltpu.sync_copy(asrc3.at[w], gbuf)
    pltpu.sync_copy(adst3.at[w], dbuf)
    pltpu.sync_copy(arel3.at[w], rbuf)
    pltpu.sync_copy(relv_hbm, relv)
    semg = (semg0, semg1)

    def start(i, b):
        pltpu.async_copy(out_nd.at[gbuf.at[i]], trows2.at[b], semg[b])
        pltpu.async_copy(out_nd.at[dbuf.at[i]], drows2.at[b], semg[b])

    def wait(b):
        pltpu.make_async_copy(out_nd.at[gbuf.at[0]], trows2.at[b],
                              semg[b]).wait()
        pltpu.make_async_copy(out_nd.at[dbuf.at[0]], drows2.at[b],
                              semg[b]).wait()

    def wait_sc(b):
        pltpu.make_async_copy(sc2.at[b], scores.at[w, pl.ds(0, C)],
                              semsc).wait()

    for b in (0, 1):
        start(b, b)

    lane = lax.iota(jnp.int32, 16)
    G = C // 16
    idx_w = [t * 16 + ((lane + t) & 15) for t in range(16)]
    idx_r = [lane * 16 + ((lane + d) & 15) for d in range(16)]

    def chunk_pair(g, _):
        for b in (0, 1):
            i = 2 * g + b
            wait(b)

            @pl.when(i >= 2)
            def _():
                wait_sc(b)

            i16 = jnp.broadcast_to(i, (16,))

            def grp_body(gi, _):
                e0 = gi * 16
                for t in range(16):
                    e = e0 + t
                    r16 = plsc.load_gather(rbuf, [i16,
                                                  jnp.broadcast_to(e, (16,))])
                    ridx = r16 * D + lane
                    a = None
                    for j in range(8):
                        sl = pl.ds(j * 16, 16)
                        rv = plsc.load_gather(relv, [ridx + j * 16])
                        pr = trows2[b, e, sl] * drows2[b, e, sl] * rv
                        a = pr if a is None else a + pr
                    plsc.store_scatter(tsc, [idx_w[t]], a)
                s = plsc.load_gather(tsc, [idx_r[0]])
                for d in range(1, 16):
                    s = s + plsc.load_gather(tsc, [idx_r[d]])
                sc2[b, pl.ds(e0, 16)] = s
                return 0

            lax.fori_loop(0, G, grp_body, 0)
            pltpu.async_copy(sc2.at[b], scores.at[w, pl.ds(i * C, C)], semsc)

            @pl.when(i + 2 < DCH)
            def _():
                start(i + 2, b)
        return 0

    lax.fori_loop(0, DCH // 2, chunk_pair, 0)
    for b in (0, 1):
        wait_sc(b)


def _decode(out_nd, asrc3, adst3, arel3, relv):
    return pl.kernel(
        _dec_body,
        out_type=jax.ShapeDtypeStruct((NW, TPWP), jnp.float32),
        mesh=_MESH,
        compiler_params=_SC_PARAMS,
        scratch_types=[pltpu.VMEM((DCH, C), jnp.int32),
                       pltpu.VMEM((DCH, C), jnp.int32),
                       pltpu.VMEM((DCH, C), jnp.int32),
                       pltpu.VMEM((R * D,), jnp.float32),
                       pltpu.VMEM((2, C, D), jnp.float32),
                       pltpu.VMEM((2, C, D), jnp.float32),
                       pltpu.VMEM((2, C), jnp.float32),
                       pltpu.VMEM((256,), jnp.float32),
                       pltpu.SemaphoreType.DMA,
                       pltpu.SemaphoreType.DMA,
                       pltpu.SemaphoreType.DMA],
    )(out_nd, asrc3, adst3, arel3, relv)


def _pad_edges(v, pad_val, per_w, per_wp):
    v2 = v.reshape(NW, per_w)
    v2 = jnp.pad(v2, ((0, 0), (0, per_wp - per_w)), constant_values=pad_val)
    return v2.reshape(NW, per_wp // C, C)


def kernel(node_embeddings, node_indices, edge_index, edge_type,
           neg_edge_index, neg_edge_type, W_proj, b_proj,
           W_rel0, W_root0, b_conv0, W_rel1, W_root1, b_conv1,
           W_rel2, W_root2, b_conv2, rel_emb):
    src = edge_index[0]
    dst = edge_index[1]
    rel = edge_type
    proj = _proj_mm(node_embeddings, W_proj, b_proj.reshape(1, D))

    g2d = _pad_edges(rel * N + src, 0, EPW, EPWP).reshape(NW, EPWP)
    dst3 = _pad_edges(dst, PAD_DST, EPW, EPWP)
    rel3 = _pad_edges(rel, 0, EPW, EPWP)
    counts = _count(dst3, rel3)
    norm3 = _norm(dst3, rel3, counts)
    dst3a = dst3.reshape(NW, NCHE, CA)
    norm3a = norm3.reshape(NW, NCHE, CA)
    zeros_nd = jnp.zeros((NP, D), jnp.float32)

    y, root = _layer_mm(proj, None, W_rel0, W_root0,
                        b_conv0.reshape(1, D), first=True)
    p = _agg(y.reshape(R * N, D), g2d, dst3a, norm3a, zeros_nd)
    y, root = _layer_mm(p, root, W_rel1, W_root1,
                        b_conv1.reshape(1, D), first=False)
    p = _agg(y.reshape(R * N, D), g2d, dst3a, norm3a, zeros_nd)
    y, root = _layer_mm(p, root, W_rel2, W_root2,
                        b_conv2.reshape(1, D), first=False)
    p = _agg(y.reshape(R * N, D), g2d, dst3a, norm3a, zeros_nd)

    out = _final_mm(p, root)

    asrc3 = _pad_edges(jnp.concatenate([src, neg_edge_index[0]]), 0, TPW, TPWP)
    arel3 = _pad_edges(jnp.concatenate([rel, neg_edge_type]), 0, TPW, TPWP)
    adst3 = _pad_edges(jnp.concatenate([dst, neg_edge_index[1]]), 0, TPW, TPWP)
    scores_p = _decode(out, asrc3, adst3, arel3, rel_emb.reshape(R * D))
    scores = scores_p[:, :TPW].reshape(E2)
    return out, scores


# speedup vs baseline: 1.2150x; 1.2150x over previous
import jax
import jax.numpy as jnp
from jax import lax
from jax.experimental import pallas as pl
from jax.experimental.pallas import tpu as pltpu
from jax.experimental.pallas import tpu_sc as plsc

N = 10000
E = 320000
R = 8
D = 128

NC = 2
NS = 16
NW = NC * NS

C = 128
EPW = E // NW
EPWP = 10240
NCH = EPWP // C
CA = 64
NCHE = EPWP // CA
E2 = 2 * E
TPW = E2 // NW
TPWP = 20480
DCH = TPWP // C
NBLK = 32
BL = E // NBLK
NR = N * R
NRP = 81920
NP = 10240
RPT = NP // NS
PAD_DST = 10200

_MESH = plsc.VectorSubcoreMesh(core_axis_name="c", subcore_axis_name="s",
                               num_cores=NC, num_subcores=NS)
_SC_PARAMS = pltpu.CompilerParams(needs_layout_passes=False)

_BN = 2000
_NB = N // _BN


def _proj_body(x_ref, w_ref, b_ref, o_ref):
    o_ref[...] = jnp.dot(x_ref[...], w_ref[...],
                         preferred_element_type=jnp.float32) + b_ref[...]


def _proj_mm(x, w, b):
    return pl.pallas_call(
        _proj_body,
        grid=(_NB,),
        in_specs=[pl.BlockSpec((_BN, D), lambda i: (i, 0)),
                  pl.BlockSpec((D, D), lambda i: (0, 0)),
                  pl.BlockSpec((1, D), lambda i: (0, 0))],
        out_specs=pl.BlockSpec((_BN, D), lambda i: (i, 0)),
        out_shape=jax.ShapeDtypeStruct((N, D), jnp.float32),
    )(x, w, b)


def _layer_first_body(x_ref, wr_ref, wo_ref, b_ref, y_ref, root_ref):
    r = pl.program_id(1)
    xb = x_ref[...]
    y_ref[...] = jnp.dot(xb, wr_ref[0],
                         preferred_element_type=jnp.float32)[None]

    @pl.when(r == 0)
    def _():
        root_ref[...] = jnp.dot(xb, wo_ref[...],
                                preferred_element_type=jnp.float32) + b_ref[...]


def _layer_mid_body(p_ref, rin_ref, wr_ref, wo_ref, b_ref, y_ref, root_ref):
    r = pl.program_id(1)
    xb = jax.nn.relu(p_ref[0] + p_ref[1] + rin_ref[...])
    y_ref[...] = jnp.dot(xb, wr_ref[0],
                         preferred_element_type=jnp.float32)[None]

    @pl.when(r == 0)
    def _():
        root_ref[...] = jnp.dot(xb, wo_ref[...],
                                preferred_element_type=jnp.float32) + b_ref[...]


def _layer_mm(x_or_p, rin, w_rel, w_root, b, first):
    specs = [
        pl.BlockSpec((D, D), lambda i, r: (0, 0)),
        pl.BlockSpec((1, D), lambda i, r: (0, 0)),
    ]
    wr_spec = pl.BlockSpec((1, D, D), lambda i, r: (r, 0, 0))
    if first:
        in_specs = [pl.BlockSpec((_BN, D), lambda i, r: (i, 0)), wr_spec] + specs
        body = _layer_first_body
        args = (x_or_p, w_rel, w_root, b)
    else:
        in_specs = [pl.BlockSpec((2, _BN, D), lambda i, r: (0, i, 0)),
                    pl.BlockSpec((_BN, D), lambda i, r: (i, 0)),
                    wr_spec] + specs
        body = _layer_mid_body
        args = (x_or_p, rin, w_rel, w_root, b)
    return pl.pallas_call(
        body,
        grid=(_NB, R),
        in_specs=in_specs,
        out_specs=[pl.BlockSpec((1, _BN, D), lambda i, r: (r, i, 0)),
                   pl.BlockSpec((_BN, D), lambda i, r: (i, 0))],
        out_shape=[jax.ShapeDtypeStruct((R, N, D), jnp.float32),
                   jax.ShapeDtypeStruct((N, D), jnp.float32)],
    )(*args)


def _final_body(p_ref, rin_ref, re_ref, out_ref, t_ref):
    r = pl.program_id(1)
    ob = p_ref[0] + p_ref[1] + rin_ref[...]
    rmask = lax.broadcasted_iota(jnp.int32, (R, D), 0) == r
    rrow = jnp.sum(jnp.where(rmask, re_ref[...], 0.0), axis=0, keepdims=True)
    t_ref[...] = (ob * rrow)[None]

    @pl.when(r == 0)
    def _():
        out_ref[...] = ob


def _final_mm(p, rin, rel_emb):
    return pl.pallas_call(
        _final_body,
        grid=(_NB, R),
        in_specs=[pl.BlockSpec((2, _BN, D), lambda i, r: (0, i, 0)),
                  pl.BlockSpec((_BN, D), lambda i, r: (i, 0)),
                  pl.BlockSpec((R, D), lambda i, r: (0, 0))],
        out_specs=[pl.BlockSpec((_BN, D), lambda i, r: (i, 0)),
                   pl.BlockSpec((1, _BN, D), lambda i, r: (r, i, 0))],
        out_shape=[jax.ShapeDtypeStruct((N, D), jnp.float32),
                   jax.ShapeDtypeStruct((R, N, D), jnp.float32)],
    )(p, rin, rel_emb)


CSL = 2560


def _count1_body(dst3, rel3, counts1, dbuf, rbuf, tbl):
    cid = lax.axis_index("c")
    sid = lax.axis_index("s")
    w = cid * NS + sid
    ones16 = jnp.full((16,), 1.0, jnp.float32)
    z16 = jnp.zeros((16,), jnp.float32)
    pltpu.sync_copy(dst3.at[w], dbuf)
    pltpu.sync_copy(rel3.at[w], rbuf)

    def zero_body(k, _):
        tbl[pl.ds(k * 16, 16)] = z16
        return 0

    lax.fori_loop(0, NRP // 16, zero_body, 0)

    def g_body(g, _):
        r = g // (C // 16)
        c = (g % (C // 16)) * 16
        d16 = dbuf[r, pl.ds(c, 16)]
        r16 = rbuf[r, pl.ds(c, 16)]
        plsc.addupdate_scatter(tbl, [d16 * R + r16], ones16)
        return 0

    lax.fori_loop(0, NCH * (C // 16), g_body, 0)
    pltpu.sync_copy(tbl, counts1.at[w])


def _count2_body(counts1, counts_out, inbuf, obuf, sem):
    cid = lax.axis_index("c")
    sid = lax.axis_index("s")
    w = cid * NS + sid

    def reduce_slice(off, ln):
        for j in range(NW):
            pltpu.async_copy(counts1.at[j, pl.ds(off, ln)],
                             inbuf.at[j, pl.ds(0, ln)], sem)
        for j in range(NW):
            pltpu.make_async_copy(counts1.at[j, pl.ds(off, ln)],
                                  inbuf.at[j, pl.ds(0, ln)], sem).wait()

        def g_body(g, _):
            sl = pl.ds(g * 16, 16)
            a = inbuf[0, sl]
            for j in range(1, NW):
                a = a + inbuf[j, sl]
            obuf[sl] = a
            return 0

        lax.fori_loop(0, ln // 16, g_body, 0)
        pltpu.sync_copy(obuf.at[pl.ds(0, ln)], counts_out.at[pl.ds(off, ln)])

    @pl.when(w < 31)
    def _():
        reduce_slice(w * CSL, CSL)

    @pl.when(w == 31)
    def _():
        reduce_slice(31 * CSL, NR - 31 * CSL)


def _count(dst3, rel3):
    counts1 = pl.kernel(
        _count1_body,
        out_type=jax.ShapeDtypeStruct((NW, NRP), jnp.float32),
        mesh=_MESH,
        compiler_params=_SC_PARAMS,
        scratch_types=[pltpu.VMEM((NCH, C), jnp.int32),
                       pltpu.VMEM((NCH, C), jnp.int32),
                       pltpu.VMEM((NRP,), jnp.float32)],
    )(dst3, rel3)
    return pl.kernel(
        _count2_body,
        out_type=jax.ShapeDtypeStruct((NR,), jnp.float32),
        mesh=_MESH,
        compiler_params=_SC_PARAMS,
        scratch_types=[pltpu.VMEM((NW, CSL), jnp.float32),
                       pltpu.VMEM((CSL,), jnp.float32),
                       pltpu.SemaphoreType.DMA],
    )(counts1)


def _norm_body(dst3, rel3, counts, norm_out, dbuf, rbuf, nbuf, tbl):
    cid = lax.axis_index("c")
    sid = lax.axis_index("s")
    w = cid * NS + sid
    pltpu.sync_copy(counts, tbl.at[pl.ds(0, NR)])
    pltpu.sync_copy(dst3.at[w], dbuf)
    pltpu.sync_copy(rel3.at[w], rbuf)

    G = C // 16

    def g_body(k, _):
        r = k // G
        c = (k % G) * 16
        d16 = dbuf[r, pl.ds(c, 16)]
        r16 = rbuf[r, pl.ds(c, 16)]
        c16 = plsc.load_gather(tbl, [d16 * R + r16])
        nbuf[r, pl.ds(c, 16)] = 1.0 / jnp.maximum(c16, 1.0)
        return 0

    lax.fori_loop(0, NCH * G, g_body, 0)
    pltpu.sync_copy(nbuf, norm_out.at[w])


def _norm(dst3, rel3, counts):
    return pl.kernel(
        _norm_body,
        out_type=jax.ShapeDtypeStruct((NW, NCH, C), jnp.float32),
        mesh=_MESH,
        compiler_params=_SC_PARAMS,
        scratch_types=[pltpu.VMEM((NCH, C), jnp.int32),
                       pltpu.VMEM((NCH, C), jnp.int32),
                       pltpu.VMEM((NCH, C), jnp.float32),
                       pltpu.VMEM((NRP,), jnp.float32)],
    )(dst3, rel3, counts)


def _agg_body(y2, g2d, dst3, norm3, zeros_nd, out,
              gbuf, d2, n2, rows4, acc,
              sg0, sg1, sg2, sg3, ss0, ss1, ss2, ss3):
    cid = lax.axis_index("c")
    sid = lax.axis_index("s")
    w = cid * NS + sid
    base = sid * RPT
    sg = (sg0, sg1, sg2, sg3)
    ss = (ss0, ss1, ss2, ss3)
    pltpu.sync_copy(zeros_nd.at[pl.ds(base, RPT)], acc.at[pl.ds(base, RPT)])
    pltpu.sync_copy(g2d.at[w], gbuf)
    plsc.subcore_barrier()

    def start_g(i, b):
        pltpu.async_copy(y2.at[gbuf.at[pl.ds(i * CA, CA)]], rows4.at[b], sg[b])
        pltpu.async_copy(dst3.at[w, i], d2.at[b], sg[b])
        pltpu.async_copy(norm3.at[w, i], n2.at[b], sg[b])

    def wait_g(b):
        pltpu.make_async_copy(y2.at[gbuf.at[pl.ds(0, CA)]], rows4.at[b],
                              sg[b]).wait()
        pltpu.make_async_copy(dst3.at[w, 0], d2.at[b], sg[b]).wait()
        pltpu.make_async_copy(norm3.at[w, 0], n2.at[b], sg[b]).wait()

    def wait_s(b):
        pltpu.make_async_copy(rows4.at[b], acc.at[d2.at[b]], ss[b]).wait()

    start_g(0, 0)
    start_g(1, 1)

    GA = CA // 16

    def chunk_quad(k, _):
        for b in range(4):
            i = 4 * k + b
            wait_g(b)
            b16 = jnp.full((16,), b, jnp.int32)

            def scale_grp(gi, _):
                e0 = gi * 16
                for t in range(16):
                    e = e0 + t
                    nv = plsc.load_gather(n2, [b16, jnp.broadcast_to(e, (16,))])
                    for j in range(8):
                        sl = pl.ds(j * 16, 16)
                        rows4[b, e, sl] = rows4[b, e, sl] * nv
                return 0

            lax.fori_loop(0, GA, scale_grp, 0)
            pltpu.async_copy(rows4.at[b], acc.at[d2.at[b]], ss[b], add=True)
            b2 = (b + 2) % 4

            @pl.when(i + 2 < NCHE)
            def _():
                @pl.when(i >= 2)
                def _():
                    wait_s(b2)

                start_g(i + 2, b2)
        return 0

    lax.fori_loop(0, NCHE // 4, chunk_quad, 0)
    for b in range(4):
        wait_s(b)
    plsc.subcore_barrier()
    pltpu.sync_copy(acc.at[pl.ds(base, RPT)], out.at[cid, pl.ds(base, RPT)])


def _agg(y2, g2d, dst3, norm3, zeros_nd):
    return pl.kernel(
        _agg_body,
        out_type=jax.ShapeDtypeStruct((NC, NP, D), jnp.float32),
        mesh=_MESH,
        compiler_params=_SC_PARAMS,
        scratch_types=[pltpu.VMEM((EPWP,), jnp.int32),
                       pltpu.VMEM((4, CA), jnp.int32),
                       pltpu.VMEM((4, CA), jnp.float32),
                       pltpu.VMEM((4, CA, D), jnp.float32),
                       pltpu.VMEM_SHARED((NP, D), jnp.float32),
                       pltpu.SemaphoreType.DMA,
                       pltpu.SemaphoreType.DMA,
                       pltpu.SemaphoreType.DMA,
                       pltpu.SemaphoreType.DMA,
                       pltpu.SemaphoreType.DMA,
                       pltpu.SemaphoreType.DMA,
                       pltpu.SemaphoreType.DMA,
                       pltpu.SemaphoreType.DMA],
    )(y2, g2d, dst3, norm3, zeros_nd)


def _dec_body(t2, out_nd, ag3, adst3, scores,
              gbuf, dbuf, trows2, drows2, sc2, tsc, semg0, semg1, semsc):
    cid = lax.axis_index("c")
    sid = lax.axis_index("s")
    w = cid * NS + sid
    pltpu.sync_copy(ag3.at[w], gbuf)
    pltpu.sync_copy(adst3.at[w], dbuf)
    semg = (semg0, semg1)

    def start(i, b):
        pltpu.async_copy(t2.at[gbuf.at[i]], trows2.at[b], semg[b])
        pltpu.async_copy(out_nd.at[dbuf.at[i]], drows2.at[b], semg[b])

    def wait(b):
        pltpu.make_async_copy(t2.at[gbuf.at[0]], trows2.at[b], semg[b]).wait()
        pltpu.make_async_copy(out_nd.at[dbuf.at[0]], drows2.at[b], semg[b]).wait()

    def wait_sc(b):
        pltpu.make_async_copy(sc2.at[b], scores.at[w, pl.ds(0, C)],
                              semsc).wait()

    for b in (0, 1):
        start(b, b)

    lane = lax.iota(jnp.int32, 16)
    G = C // 16
    idx_w = [t * 16 + ((lane + t) & 15) for t in range(16)]
    idx_r = [lane * 16 + ((lane + d) & 15) for d in range(16)]

    def chunk_pair(g, _):
        for b in (0, 1):
            i = 2 * g + b
            wait(b)

            @pl.when(i >= 2)
            def _():
                wait_sc(b)

            def grp_body(gi, _):
                e0 = gi * 16
                for t in range(16):
                    e = e0 + t
                    a = trows2[b, e, pl.ds(0, 16)] * drows2[b, e, pl.ds(0, 16)]
                    for j in range(1, 8):
                        sl = pl.ds(j * 16, 16)
                        a = a + trows2[b, e, sl] * drows2[b, e, sl]
                    plsc.store_scatter(tsc, [idx_w[t]], a)
                s = plsc.load_gather(tsc, [idx_r[0]])
                for d in range(1, 16):
                    s = s + plsc.load_gather(tsc, [idx_r[d]])
                sc2[b, pl.ds(e0, 16)] = s
                return 0

            lax.fori_loop(0, G, grp_body, 0)
            pltpu.async_copy(sc2.at[b], scores.at[w, pl.ds(i * C, C)], semsc)

            @pl.when(i + 2 < DCH)
            def _():
                start(i + 2, b)
        return 0

    lax.fori_loop(0, DCH // 2, chunk_pair, 0)
    for b in (0, 1):
        wait_sc(b)


def _decode(t2, out_nd, ag3, adst3):
    return pl.kernel(
        _dec_body,
        out_type=jax.ShapeDtypeStruct((NW, TPWP), jnp.float32),
        mesh=_MESH,
        compiler_params=_SC_PARAMS,
        scratch_types=[pltpu.VMEM((DCH, C), jnp.int32),
                       pltpu.VMEM((DCH, C), jnp.int32),
                       pltpu.VMEM((2, C, D), jnp.float32),
                       pltpu.VMEM((2, C, D), jnp.float32),
                       pltpu.VMEM((2, C), jnp.float32),
                       pltpu.VMEM((256,), jnp.float32),
                       pltpu.SemaphoreType.DMA,
                       pltpu.SemaphoreType.DMA,
                       pltpu.SemaphoreType.DMA],
    )(t2, out_nd, ag3, adst3)


def _pad_edges(v, pad_val, per_w, per_wp):
    v2 = v.reshape(NW, per_w)
    v2 = jnp.pad(v2, ((0, 0), (0, per_wp - per_w)), constant_values=pad_val)
    return v2.reshape(NW, per_wp // C, C)


def kernel(node_embeddings, node_indices, edge_index, edge_type,
           neg_edge_index, neg_edge_type, W_proj, b_proj,
           W_rel0, W_root0, b_conv0, W_rel1, W_root1, b_conv1,
           W_rel2, W_root2, b_conv2, rel_emb):
    src = edge_index[0]
    dst = edge_index[1]
    rel = edge_type
    proj = _proj_mm(node_embeddings, W_proj, b_proj.reshape(1, D))

    g2d = _pad_edges(rel * N + src, 0, EPW, EPWP).reshape(NW, EPWP)
    dst3 = _pad_edges(dst, PAD_DST, EPW, EPWP)
    rel3 = _pad_edges(rel, 0, EPW, EPWP)
    counts = _count(dst3, rel3)
    norm3 = _norm(dst3, rel3, counts)
    dst3a = dst3.reshape(NW, NCHE, CA)
    norm3a = norm3.reshape(NW, NCHE, CA)
    zeros_nd = jnp.zeros((NP, D), jnp.float32)

    y, root = _layer_mm(proj, None, W_rel0, W_root0,
                        b_conv0.reshape(1, D), first=True)
    p = _agg(y.reshape(R * N, D), g2d, dst3a, norm3a, zeros_nd)
    y, root = _layer_mm(p, root, W_rel1, W_root1,
                        b_conv1.reshape(1, D), first=False)
    p = _agg(y.reshape(R * N, D), g2d, dst3a, norm3a, zeros_nd)
    y, root = _layer_mm(p, root, W_rel2, W_root2,
                        b_conv2.reshape(1, D), first=False)
    p = _agg(y.reshape(R * N, D), g2d, dst3a, norm3a, zeros_nd)

    out, t = _final_mm(p, root, rel_emb)

    all_src = jnp.concatenate([src, neg_edge_index[0]])
    all_rel = jnp.concatenate([rel, neg_edge_type])
    ag3 = _pad_edges(all_rel * N + all_src, 0, TPW, TPWP)
    adst3 = _pad_edges(jnp.concatenate([dst, neg_edge_index[1]]), 0, TPW, TPWP)
    scores_p = _decode(t.reshape(R * N, D), out, ag3, adst3)
    scores = scores_p[:, :TPW].reshape(E2)
    return out, scores
